# Initial kernel scaffold; baseline (speedup 1.0000x reference)
import jax
import jax.numpy as jnp
from jax.experimental import pallas as pl


def kernel(x, batch):
    raise NotImplementedError("write your pallas kernel here")


# speedup vs baseline: 3.6877x; 3.6877x over previous
import functools

import jax
import jax.numpy as jnp
from jax import lax
from jax.experimental import pallas as pl
from jax.experimental.pallas import tpu as pltpu
from jax.experimental.pallas import tpu_sc as plsc

N_NODES = 100000
D = 128
NSEG = 256
NW = 32
CHUNK = 128
CPW = 25
RPW = CHUNK * CPW
NPAD = NW * RPW
TAIL = N_NODES % CHUNK
L = 16
NJ = D // L
SUB = 16


def _sc_partial_sums(x_flat, batch_flat):
    mesh = plsc.VectorSubcoreMesh(core_axis_name="c", subcore_axis_name="s")

    @functools.partial(
        pl.kernel,
        mesh=mesh,
        out_type=jax.ShapeDtypeStruct((NW, (NSEG + 1) * D), jnp.float32),
        scratch_types=[
            pltpu.VMEM((RPW,), jnp.int32),
            pltpu.VMEM((CHUNK * D,), jnp.float32),
            pltpu.VMEM(((NSEG + 1) * D,), jnp.float32),
        ],
    )
    def k(x_hbm, b_hbm, out_hbm, idx_v, rows_v, acc_v):
        wid = lax.axis_index("s") * 2 + lax.axis_index("c")
        zero16 = jnp.zeros((L,), jnp.float32)

        def zrow(i, carry):
            acc_v[pl.ds(i * L, L)] = zero16
            return carry

        lax.fori_loop(0, (NSEG + 1) * D // L, zrow, 0)

        pltpu.sync_copy(b_hbm.at[pl.ds(wid * RPW, RPW)], idx_v)

        def row_step(segs, r, row0, state):
            seg_cur, base_cur, regs = state
            for j in range(NJ):
                acc_v[pl.ds(base_cur + L * j, L)] = regs[j]
            s = segs[r]
            keep = jnp.where(s == seg_cur, 1.0, 0.0).astype(jnp.float32)
            keep_v = jnp.broadcast_to(keep, (L,))
            regs = tuple(
                regs[j] * keep_v + rows_v[pl.ds(row0 + r * D + L * j, L)]
                for j in range(NJ)
            )
            return s, s * D, regs

        def chunk_body(c, state):
            row_base = wid * RPW + c * CHUNK

            @pl.when(row_base + CHUNK <= N_NODES)
            def _full():
                pltpu.sync_copy(x_hbm.at[pl.ds(row_base * D, CHUNK * D)],
                                rows_v)

            @pl.when(jnp.logical_and(row_base < N_NODES,
                                     row_base + CHUNK > N_NODES))
            def _tail():
                pltpu.sync_copy(x_hbm.at[pl.ds(row_base * D, TAIL * D)],
                                rows_v.at[pl.ds(0, TAIL * D)])

            def sub_body(b, st):
                segs = idx_v[pl.ds(c * CHUNK + b * SUB, SUB)]
                for r in range(SUB):
                    st = row_step(segs, r, b * SUB * D, st)
                return st

            return lax.fori_loop(0, CHUNK // SUB, sub_body, state)

        seg0 = idx_v[pl.ds(0, L)][0]
        regs0 = tuple(zero16 for _ in range(NJ))
        state0 = (seg0, seg0 * D, regs0)
        _, base_f, regs_f = lax.fori_loop(0, CPW, chunk_body, state0)
        for j in range(NJ):
            acc_v[pl.ds(base_f + L * j, L)] = regs_f[j]

        pltpu.sync_copy(acc_v, out_hbm.at[wid])

    return k(x_flat, batch_flat)


def _tc_reduce(partials):
    def body(p_ref, o_ref):
        p = p_ref[...].reshape(NW, NSEG + 1, D)
        o_ref[...] = jnp.sum(p[:, :NSEG, :], axis=0)

    return pl.pallas_call(
        body,
        out_shape=jax.ShapeDtypeStruct((NSEG, D), jnp.float32),
    )(partials)


def kernel(x, batch):
    pad = jnp.full((NPAD - N_NODES,), NSEG, jnp.int32)
    b_flat = jnp.concatenate([batch, pad])
    partials = _sc_partial_sums(x.reshape(-1), b_flat)
    return _tc_reduce(partials)


# speedup vs baseline: 5.6119x; 1.5218x over previous
import functools

import jax
import jax.numpy as jnp
from jax import lax
from jax.experimental import pallas as pl
from jax.experimental.pallas import tpu as pltpu
from jax.experimental.pallas import tpu_sc as plsc

N_NODES = 100000
D = 128
NSEG = 256
NW = 32
CHUNK = 160
CPW = 20
RPW = CHUNK * CPW
NPAD = NW * RPW
L = 16
NJ = D // L
SUB = 16
GPC = CHUNK // SUB


def _sc_partial_sums(x_flat, batch_flat):
    mesh = plsc.VectorSubcoreMesh(core_axis_name="c", subcore_axis_name="s")

    @functools.partial(
        pl.kernel,
        mesh=mesh,
        out_type=jax.ShapeDtypeStruct((NW, (NSEG + 1) * D), jnp.float32),
        scratch_types=[
            pltpu.VMEM((RPW,), jnp.int32),
            pltpu.VMEM((CHUNK * D,), jnp.float32),
            pltpu.VMEM((CHUNK * D,), jnp.float32),
            pltpu.VMEM(((NSEG + 1) * D,), jnp.float32),
            pltpu.SemaphoreType.DMA,
            pltpu.SemaphoreType.DMA,
        ],
    )
    def k(x_hbm, b_hbm, out_hbm, idx_v, rows_a, rows_b, acc_v,
          sem_a, sem_b):
        wid = lax.axis_index("s") * 2 + lax.axis_index("c")
        zero16 = jnp.zeros((L,), jnp.float32)
        slab = wid * RPW

        def zrow(i, carry):
            for j in range(NJ):
                acc_v[pl.ds(i * D + L * j, L)] = zero16
            return carry

        lax.fori_loop(0, NSEG + 1, zrow, 0)

        pltpu.sync_copy(b_hbm.at[pl.ds(slab, RPW)], idx_v)

        def chunk_dma(c, buf, sem):
            row_base = slab + c * CHUNK
            return pltpu.make_async_copy(
                x_hbm.at[pl.ds(row_base * D, CHUNK * D)], buf, sem)

        def is_real(c):
            return slab + c * CHUNK + CHUNK <= N_NODES

        def group_body(c, rows_v, g, carry):
            segs = idx_v[pl.ds(c * CHUNK + g * SUB, SUB)]
            first = segs[0]
            last = segs[SUB - 1]
            row0 = g * SUB * D

            @pl.when(first == last)
            def _uniform():
                base = first * D
                for j in range(NJ):
                    vs = [rows_v[pl.ds(row0 + r * D + L * j, L)]
                          for r in range(SUB)]
                    while len(vs) > 1:
                        vs = [vs[i] + vs[i + 1]
                              for i in range(0, len(vs), 2)]
                    plsc.addupdate(acc_v.at[pl.ds(base + L * j, L)], vs[0])

            @pl.when(first != last)
            def _mixed():
                for r in range(SUB):
                    base = segs[r] * D
                    for j in range(NJ):
                        v = rows_v[pl.ds(row0 + r * D + L * j, L)]
                        plsc.addupdate(
                            acc_v.at[pl.ds(base + L * j, L)], v)

            return carry

        def pair_body(cp, carry):
            for b, (buf, sem) in enumerate(((rows_a, sem_a),
                                            (rows_b, sem_b))):
                c = 2 * cp + b

                @pl.when(is_real(c))
                def _wait():
                    chunk_dma(c, buf, sem).wait()

                def gb(g, st):
                    return group_body(c, buf, g, st)

                carry = lax.fori_loop(0, GPC, gb, carry)

                @pl.when(jnp.logical_and(is_real(c + 2), c + 2 < CPW))
                def _start():
                    chunk_dma(c + 2, buf, sem).start()

            return carry

        @pl.when(is_real(0))
        def _p0():
            chunk_dma(0, rows_a, sem_a).start()

        @pl.when(is_real(1))
        def _p1():
            chunk_dma(1, rows_b, sem_b).start()

        lax.fori_loop(0, CPW // 2, pair_body, 0)

        pltpu.sync_copy(acc_v, out_hbm.at[wid])

    return k(x_flat, batch_flat)


def _tc_reduce(partials):
    def body(p_ref, o_ref):
        p = p_ref[...].reshape(NW, NSEG + 1, D)
        o_ref[...] = jnp.sum(p[:, :NSEG, :], axis=0)

    return pl.pallas_call(
        body,
        out_shape=jax.ShapeDtypeStruct((NSEG, D), jnp.float32),
    )(partials)


def kernel(x, batch):
    pad = jnp.full((NPAD - N_NODES,), NSEG, jnp.int32)
    b_flat = jnp.concatenate([batch, pad])
    partials = _sc_partial_sums(x.reshape(-1), b_flat)
    return _tc_reduce(partials)
